# Initial kernel scaffold; baseline (speedup 1.0000x reference)
import jax
import jax.numpy as jnp
from jax.experimental import pallas as pl


def kernel(src_tokens, padded_coordinates, src_distance, src_edge_type, embed_table, rbf_centers, edge_W, edge_b, up_W1, up_b1, up_W2, up_b2):
    raise NotImplementedError("write your pallas kernel here")


# speedup vs baseline: 271.2148x; 271.2148x over previous
import functools

import jax
import jax.numpy as jnp
from jax import lax
from jax.experimental import pallas as pl

VOCAB = 128
PAD = 0
RBF_DIM = 64
CUTOFF = 6.0
GAMMA = 10.0


def _sdeg_kernel(ccol_ref, crow_ref, cen_ref, out_ref):
    ci = ccol_ref[0]
    cix = ci[:, 0:1]
    ciy = ci[:, 1:2]
    ciz = ci[:, 2:3]
    mi = ci[:, 3:4]
    cr = crow_ref[0]
    cjx = cr[0:1, :]
    cjy = cr[1:2, :]
    cjz = cr[2:3, :]
    mj = cr[3:4, :]
    n = ci.shape[0]

    dx = cix - cjx
    dy = ciy - cjy
    dz = ciz - cjz
    d2diff = dx * dx + dy * dy + dz * dz
    dist = jnp.sqrt(d2diff)

    sqi = cix * cix + ciy * ciy + ciz * ciz
    sqj = cjx * cjx + cjy * cjy + cjz * cjz
    bf = jnp.bfloat16
    bix = cix.astype(bf).astype(jnp.float32)
    biy = ciy.astype(bf).astype(jnp.float32)
    biz = ciz.astype(bf).astype(jnp.float32)
    bjx = cjx.astype(bf).astype(jnp.float32)
    bjy = cjy.astype(bf).astype(jnp.float32)
    bjz = cjz.astype(bf).astype(jnp.float32)
    d2m = (sqi + sqj) - 2.0 * (bix * bjx + biy * bjy + biz * bjz)
    ii = lax.broadcasted_iota(jnp.int32, (n, n), 0)
    jj = lax.broadcasted_iota(jnp.int32, (n, n), 1)
    valid = (d2m <= CUTOFF * CUTOFF) & (ii != jj) & (mi > 0.0) & (mj > 0.0)
    w = jnp.where(valid, 1.0, 0.0).astype(jnp.float32)

    out_ref[0, :, 64:65] = jnp.sum(w, axis=1, keepdims=True)
    for k in range(RBF_DIM):
        c = cen_ref[0, k]
        t = dist - c
        e = jnp.exp(t * t * (-GAMMA))
        out_ref[0, :, k : k + 1] = jnp.sum(w * e, axis=1, keepdims=True)


def _mlp_kernel(sp_ref, tok_ref, emb_ref, ew_ref, eb_ref, w1_ref, b1_ref,
                w2_ref, b2_ref, out_ref):
    rows = sp_ref.shape[0]
    layers = ew_ref.shape[0]
    tok = tok_ref[:, :]
    vocab_ids = lax.broadcasted_iota(jnp.int32, (rows, VOCAB), 1).astype(jnp.float32)
    onehot = (tok == vocab_ids)
    x = jnp.dot(onehot.astype(jnp.float32), emb_ref[:, :],
                preferred_element_type=jnp.float32)
    s = sp_ref[:, :RBF_DIM]
    deg = sp_ref[:, RBF_DIM : RBF_DIM + 1]
    for l in range(layers):
        agg = (jnp.dot(s, ew_ref[l], preferred_element_type=jnp.float32)
               + deg * eb_ref[l : l + 1, :])
        h = jnp.dot(agg, w1_ref[l], preferred_element_type=jnp.float32)
        h = jnp.maximum(h + b1_ref[l : l + 1, :], 0.0)
        h = (jnp.dot(h, w2_ref[l], preferred_element_type=jnp.float32)
             + b2_ref[l : l + 1, :])
        x = x + h
    x = jnp.where(tok != float(PAD), x, 0.0)
    out_ref[:, :] = x


@functools.partial(jax.jit, static_argnames=())
def kernel(src_tokens, padded_coordinates, src_distance, src_edge_type,
           embed_table, rbf_centers, edge_W, edge_b, up_W1, up_b1, up_W2,
           up_b2):
    B, N = src_tokens.shape
    E = embed_table.shape[1]
    L = edge_W.shape[0]
    f32 = jnp.float32

    maskf = (src_tokens != PAD).astype(f32)
    coords = padded_coordinates.astype(f32)

    ccol = jnp.zeros((B, N, 128), f32)
    ccol = ccol.at[:, :, :3].set(coords).at[:, :, 3].set(maskf)
    crow = jnp.zeros((B, 8, N), f32)
    crow = crow.at[:, :3, :].set(jnp.swapaxes(coords, 1, 2))
    crow = crow.at[:, 3, :].set(maskf)
    cen = jnp.zeros((8, 128), f32).at[0, :RBF_DIM].set(rbf_centers.astype(f32))

    sp = pl.pallas_call(
        _sdeg_kernel,
        grid=(B,),
        in_specs=[
            pl.BlockSpec((1, N, 128), lambda b: (b, 0, 0)),
            pl.BlockSpec((1, 8, N), lambda b: (b, 0, 0)),
            pl.BlockSpec((8, 128), lambda b: (0, 0)),
        ],
        out_specs=pl.BlockSpec((1, N, 128), lambda b: (b, 0, 0)),
        out_shape=jax.ShapeDtypeStruct((B, N, 128), f32),
    )(ccol, crow, cen)

    sp_flat = sp.reshape(B * N, 128)
    tokf = src_tokens.astype(f32).reshape(B * N, 1)
    ew64 = edge_W[:, :RBF_DIM, :].astype(f32)

    BR = 256
    nblk = (B * N) // BR
    xout = pl.pallas_call(
        _mlp_kernel,
        grid=(nblk,),
        in_specs=[
            pl.BlockSpec((BR, 128), lambda i: (i, 0)),
            pl.BlockSpec((BR, 1), lambda i: (i, 0)),
            pl.BlockSpec((VOCAB, E), lambda i: (0, 0)),
            pl.BlockSpec((L, RBF_DIM, E), lambda i: (0, 0, 0)),
            pl.BlockSpec((L, E), lambda i: (0, 0)),
            pl.BlockSpec((L, E, E), lambda i: (0, 0, 0)),
            pl.BlockSpec((L, E), lambda i: (0, 0)),
            pl.BlockSpec((L, E, E), lambda i: (0, 0, 0)),
            pl.BlockSpec((L, E), lambda i: (0, 0)),
        ],
        out_specs=pl.BlockSpec((BR, E), lambda i: (i, 0)),
        out_shape=jax.ShapeDtypeStruct((B * N, E), f32),
    )(sp_flat, tokf, embed_table.astype(f32), ew64, edge_b.astype(f32),
      up_W1.astype(f32), up_b1.astype(f32), up_W2.astype(f32),
      up_b2.astype(f32))

    encoder_rep = xout.reshape(B, N, E)
    padding_mask = src_tokens == PAD
    return (encoder_rep, padding_mask)


# speedup vs baseline: 307.7855x; 1.1348x over previous
import functools

import jax
import jax.numpy as jnp
from jax import lax
from jax.experimental import pallas as pl

VOCAB = 128
PAD = 0
RBF_DIM = 64
CUTOFF = 6.0
GAMMA = 10.0


def _sdeg_kernel(ccol_ref, crow_ref, cen_ref, out_ref):
    ci = ccol_ref[0]
    cix = ci[:, 0:1]
    ciy = ci[:, 1:2]
    ciz = ci[:, 2:3]
    mi = ci[:, 3:4]
    cr = crow_ref[0]
    cjx = cr[0:1, :]
    cjy = cr[1:2, :]
    cjz = cr[2:3, :]
    mj = cr[3:4, :]
    n = ci.shape[0]

    dx = cix - cjx
    dy = ciy - cjy
    dz = ciz - cjz
    d2diff = dx * dx + dy * dy + dz * dz
    dist = jnp.sqrt(d2diff)

    sqi = cix * cix + ciy * ciy + ciz * ciz
    sqj = cjx * cjx + cjy * cjy + cjz * cjz
    bf = jnp.bfloat16
    bix = cix.astype(bf).astype(jnp.float32)
    biy = ciy.astype(bf).astype(jnp.float32)
    biz = ciz.astype(bf).astype(jnp.float32)
    bjx = cjx.astype(bf).astype(jnp.float32)
    bjy = cjy.astype(bf).astype(jnp.float32)
    bjz = cjz.astype(bf).astype(jnp.float32)
    d2m = (sqi + sqj) - 2.0 * (bix * bjx + biy * bjy + biz * bjz)
    ii = lax.broadcasted_iota(jnp.int32, (n, n), 0)
    jj = lax.broadcasted_iota(jnp.int32, (n, n), 1)
    valid = (d2m <= CUTOFF * CUTOFF) & (ii != jj) & (mi > 0.0) & (mj > 0.0)
    w = jnp.where(valid, 1.0, 0.0).astype(jnp.float32)

    LOG2E = 1.4426950408889634
    dm = jnp.where(valid, dist, 1e4)
    A = dm * dm * (-GAMMA * LOG2E)
    Bv = dm * (2.0 * GAMMA * LOG2E)
    cols = []
    for k in range(RBF_DIM):
        c = cen_ref[0, k]
        s_c = c * c * (-GAMMA * LOG2E)
        e = jnp.exp2(Bv * c + A + s_c)
        cols.append(jnp.sum(e, axis=1, keepdims=True))
    cols.append(jnp.sum(w, axis=1, keepdims=True))
    out_ref[0, :, : RBF_DIM + 1] = jnp.concatenate(cols, axis=1)


def _mlp_kernel(sp_ref, tok_ref, emb_ref, ew_ref, eb_ref, w1_ref, b1_ref,
                w2_ref, b2_ref, out_ref):
    rows = sp_ref.shape[0]
    layers = ew_ref.shape[0]
    tok = tok_ref[:, :]
    vocab_ids = lax.broadcasted_iota(jnp.int32, (rows, VOCAB), 1).astype(jnp.float32)
    onehot = (tok == vocab_ids)
    x = jnp.dot(onehot.astype(jnp.float32), emb_ref[:, :],
                preferred_element_type=jnp.float32)
    s = sp_ref[:, :RBF_DIM]
    deg = sp_ref[:, RBF_DIM : RBF_DIM + 1]
    for l in range(layers):
        agg = (jnp.dot(s, ew_ref[l], preferred_element_type=jnp.float32)
               + deg * eb_ref[l : l + 1, :])
        h = jnp.dot(agg, w1_ref[l], preferred_element_type=jnp.float32)
        h = jnp.maximum(h + b1_ref[l : l + 1, :], 0.0)
        h = (jnp.dot(h, w2_ref[l], preferred_element_type=jnp.float32)
             + b2_ref[l : l + 1, :])
        x = x + h
    x = jnp.where(tok != float(PAD), x, 0.0)
    out_ref[:, :] = x


@functools.partial(jax.jit, static_argnames=())
def kernel(src_tokens, padded_coordinates, src_distance, src_edge_type,
           embed_table, rbf_centers, edge_W, edge_b, up_W1, up_b1, up_W2,
           up_b2):
    B, N = src_tokens.shape
    E = embed_table.shape[1]
    L = edge_W.shape[0]
    f32 = jnp.float32

    maskf = (src_tokens != PAD).astype(f32)
    coords = padded_coordinates.astype(f32)

    ccol = jnp.zeros((B, N, 128), f32)
    ccol = ccol.at[:, :, :3].set(coords).at[:, :, 3].set(maskf)
    crow = jnp.zeros((B, 8, N), f32)
    crow = crow.at[:, :3, :].set(jnp.swapaxes(coords, 1, 2))
    crow = crow.at[:, 3, :].set(maskf)
    cen = jnp.zeros((8, 128), f32).at[0, :RBF_DIM].set(rbf_centers.astype(f32))

    sp = pl.pallas_call(
        _sdeg_kernel,
        grid=(B,),
        in_specs=[
            pl.BlockSpec((1, N, 128), lambda b: (b, 0, 0)),
            pl.BlockSpec((1, 8, N), lambda b: (b, 0, 0)),
            pl.BlockSpec((8, 128), lambda b: (0, 0)),
        ],
        out_specs=pl.BlockSpec((1, N, 128), lambda b: (b, 0, 0)),
        out_shape=jax.ShapeDtypeStruct((B, N, 128), f32),
    )(ccol, crow, cen)

    sp_flat = sp.reshape(B * N, 128)
    tokf = src_tokens.astype(f32).reshape(B * N, 1)
    ew64 = edge_W[:, :RBF_DIM, :].astype(f32)

    BR = 256
    nblk = (B * N) // BR
    xout = pl.pallas_call(
        _mlp_kernel,
        grid=(nblk,),
        in_specs=[
            pl.BlockSpec((BR, 128), lambda i: (i, 0)),
            pl.BlockSpec((BR, 1), lambda i: (i, 0)),
            pl.BlockSpec((VOCAB, E), lambda i: (0, 0)),
            pl.BlockSpec((L, RBF_DIM, E), lambda i: (0, 0, 0)),
            pl.BlockSpec((L, E), lambda i: (0, 0)),
            pl.BlockSpec((L, E, E), lambda i: (0, 0, 0)),
            pl.BlockSpec((L, E), lambda i: (0, 0)),
            pl.BlockSpec((L, E, E), lambda i: (0, 0, 0)),
            pl.BlockSpec((L, E), lambda i: (0, 0)),
        ],
        out_specs=pl.BlockSpec((BR, E), lambda i: (i, 0)),
        out_shape=jax.ShapeDtypeStruct((B * N, E), f32),
    )(sp_flat, tokf, embed_table.astype(f32), ew64, edge_b.astype(f32),
      up_W1.astype(f32), up_b1.astype(f32), up_W2.astype(f32),
      up_b2.astype(f32))

    encoder_rep = xout.reshape(B, N, E)
    padding_mask = src_tokens == PAD
    return (encoder_rep, padding_mask)


# speedup vs baseline: 336.6308x; 1.0937x over previous
import jax
import jax.numpy as jnp
from jax import lax
from jax.experimental import pallas as pl

VOCAB = 128
PAD = 0
RBF_DIM = 64
CUTOFF = 6.0
GAMMA = 10.0
LOG2E = 1.4426950408889634
BLK = 128


def _fused_kernel(coord_ref, tok_ref, cen_ref, emb_ref, ew_ref, eb_ref,
                  w1_ref, b1_ref, w2_ref, b2_ref, out_ref):
    n = coord_ref.shape[1]
    nb = n // BLK
    f32 = jnp.float32

    cx = coord_ref[0, :, 0:1]
    cy = coord_ref[0, :, 1:2]
    cz = coord_ref[0, :, 2:3]
    tok = tok_ref[0, :, :]
    mcol = jnp.where(tok != float(PAD), 1.0, 0.0).astype(f32)

    c8 = jnp.concatenate(
        [cx, cy, cz, mcol, jnp.zeros((n, 4), f32)], axis=1)
    r8 = c8.T
    rx = r8[0:1, :]
    ry = r8[1:2, :]
    rz = r8[2:3, :]
    mrow = r8[3:4, :]

    bf = jnp.bfloat16
    bcx = cx.astype(bf).astype(f32)
    bcy = cy.astype(bf).astype(f32)
    bcz = cz.astype(bf).astype(f32)
    brx = rx.astype(bf).astype(f32)
    bry = ry.astype(bf).astype(f32)
    brz = rz.astype(bf).astype(f32)
    sqc = cx * cx + cy * cy + cz * cz
    sqr = rx * rx + ry * ry + rz * rz

    racc = [None] * nb
    cacc = [None] * nb

    for bi in range(nb):
        i0 = bi * BLK
        cix = cx[i0:i0 + BLK]
        ciy = cy[i0:i0 + BLK]
        ciz = cz[i0:i0 + BLK]
        bix = bcx[i0:i0 + BLK]
        biy = bcy[i0:i0 + BLK]
        biz = bcz[i0:i0 + BLK]
        sqi = sqc[i0:i0 + BLK]
        mi = mcol[i0:i0 + BLK]
        for bj in range(bi, nb):
            j0 = bj * BLK
            cjx = rx[:, j0:j0 + BLK]
            cjy = ry[:, j0:j0 + BLK]
            cjz = rz[:, j0:j0 + BLK]
            bjx = brx[:, j0:j0 + BLK]
            bjy = bry[:, j0:j0 + BLK]
            bjz = brz[:, j0:j0 + BLK]
            sqj = sqr[:, j0:j0 + BLK]
            mj = mrow[:, j0:j0 + BLK]

            dx = cix - cjx
            dy = ciy - cjy
            dz = ciz - cjz
            d2diff = dx * dx + dy * dy + dz * dz
            dist = jnp.sqrt(d2diff)
            d2m = (sqi + sqj) - 2.0 * (bix * bjx + biy * bjy + biz * bjz)
            valid = (d2m <= CUTOFF * CUTOFF) & (mi > 0.0) & (mj > 0.0)
            if bi == bj:
                ii = lax.broadcasted_iota(jnp.int32, (BLK, BLK), 0)
                jj = lax.broadcasted_iota(jnp.int32, (BLK, BLK), 1)
                valid = valid & (ii != jj)
            w = jnp.where(valid, 1.0, 0.0).astype(f32)

            dm = jnp.where(valid, dist, 1e4)
            A = dm * dm * (-GAMMA * LOG2E)
            Bv = dm * (2.0 * GAMMA * LOG2E)
            rcols = []
            crows = []
            for k in range(RBF_DIM):
                c = cen_ref[0, k]
                s_c = c * c * (-GAMMA * LOG2E)
                e = jnp.exp2(Bv * c + A + s_c)
                rcols.append(jnp.sum(e, axis=1, keepdims=True))
                if bi != bj:
                    crows.append(jnp.sum(e, axis=0, keepdims=True))
            rcols.append(jnp.sum(w, axis=1, keepdims=True))
            rblk = jnp.concatenate(rcols, axis=1)
            racc[bi] = rblk if racc[bi] is None else racc[bi] + rblk
            if bi != bj:
                crows.append(jnp.sum(w, axis=0, keepdims=True))
                cblk = jnp.concatenate(crows, axis=0)
                cacc[bj] = cblk if cacc[bj] is None else cacc[bj] + cblk

    srow = jnp.concatenate(racc, axis=0)
    zpad = jnp.zeros((RBF_DIM + 1, BLK), f32)
    ct = jnp.concatenate(
        [zpad if c is None else c for c in cacc], axis=1)
    ctt = jnp.concatenate(
        [ct, jnp.zeros((BLK - RBF_DIM - 1, n), f32)], axis=0)
    stot = srow + ctt.T[:, : RBF_DIM + 1]

    s = stot[:, :RBF_DIM]
    deg = stot[:, RBF_DIM : RBF_DIM + 1]

    vocab_ids = lax.broadcasted_iota(jnp.int32, (n, VOCAB), 1).astype(f32)
    onehot = (tok == vocab_ids).astype(f32)
    x = jnp.dot(onehot, emb_ref[:, :], preferred_element_type=f32)
    layers = ew_ref.shape[0]
    for l in range(layers):
        agg = (jnp.dot(s, ew_ref[l], preferred_element_type=f32)
               + deg * eb_ref[l : l + 1, :])
        h = jnp.dot(agg, w1_ref[l], preferred_element_type=f32)
        h = jnp.maximum(h + b1_ref[l : l + 1, :], 0.0)
        h = (jnp.dot(h, w2_ref[l], preferred_element_type=f32)
             + b2_ref[l : l + 1, :])
        x = x + h
    x = jnp.where(tok != float(PAD), x, 0.0)
    out_ref[0, :, :] = x


def kernel(src_tokens, padded_coordinates, src_distance, src_edge_type,
           embed_table, rbf_centers, edge_W, edge_b, up_W1, up_b1, up_W2,
           up_b2):
    B, N = src_tokens.shape
    E = embed_table.shape[1]
    L = edge_W.shape[0]
    f32 = jnp.float32

    tokf = src_tokens.astype(f32).reshape(B, N, 1)
    coords = padded_coordinates.astype(f32)
    cen = jnp.zeros((8, 128), f32).at[0, :RBF_DIM].set(rbf_centers.astype(f32))
    ew64 = edge_W[:, :RBF_DIM, :].astype(f32)

    xout = pl.pallas_call(
        _fused_kernel,
        grid=(B,),
        in_specs=[
            pl.BlockSpec((1, N, 3), lambda b: (b, 0, 0)),
            pl.BlockSpec((1, N, 1), lambda b: (b, 0, 0)),
            pl.BlockSpec((8, 128), lambda b: (0, 0)),
            pl.BlockSpec((VOCAB, E), lambda b: (0, 0)),
            pl.BlockSpec((L, RBF_DIM, E), lambda b: (0, 0, 0)),
            pl.BlockSpec((L, E), lambda b: (0, 0)),
            pl.BlockSpec((L, E, E), lambda b: (0, 0, 0)),
            pl.BlockSpec((L, E), lambda b: (0, 0)),
            pl.BlockSpec((L, E, E), lambda b: (0, 0, 0)),
            pl.BlockSpec((L, E), lambda b: (0, 0)),
        ],
        out_specs=pl.BlockSpec((1, N, E), lambda b: (b, 0, 0)),
        out_shape=jax.ShapeDtypeStruct((B, N, E), f32),
    )(coords, tokf, cen, embed_table.astype(f32), ew64, edge_b.astype(f32),
      up_W1.astype(f32), up_b1.astype(f32), up_W2.astype(f32),
      up_b2.astype(f32))

    padding_mask = src_tokens == PAD
    return (xout, padding_mask)


# speedup vs baseline: 489.9101x; 1.4553x over previous
import jax
import jax.numpy as jnp
from jax import lax
from jax.experimental import pallas as pl

VOCAB = 128
PAD = 0
RBF_DIM = 64
CUTOFF = 6.0
GAMMA = 10.0
LOG2E = 1.4426950408889634
BLK = 128


def _fused_kernel(coord_ref, tok_ref, cen_ref, emb_ref, ew_ref, eb_ref,
                  w1_ref, b1_ref, w2_ref, b2_ref, out_ref):
    n = coord_ref.shape[1]
    nb = n // BLK
    f32 = jnp.float32

    cx = coord_ref[0, :, 0:1]
    cy = coord_ref[0, :, 1:2]
    cz = coord_ref[0, :, 2:3]
    tok = tok_ref[0, :, :]
    mcol = jnp.where(tok != float(PAD), 1.0, 0.0).astype(f32)

    c8 = jnp.concatenate(
        [cx, cy, cz, mcol, jnp.zeros((n, 4), f32)], axis=1)
    r8 = c8.T
    rx = r8[0:1, :]
    ry = r8[1:2, :]
    rz = r8[2:3, :]
    mrow = r8[3:4, :]

    bf = jnp.bfloat16
    bcx = cx.astype(bf).astype(f32)
    bcy = cy.astype(bf).astype(f32)
    bcz = cz.astype(bf).astype(f32)
    brx = rx.astype(bf).astype(f32)
    bry = ry.astype(bf).astype(f32)
    brz = rz.astype(bf).astype(f32)
    sqc = cx * cx + cy * cy + cz * cz
    sqr = rx * rx + ry * ry + rz * rz

    cen_sc = []
    for k in range(RBF_DIM):
        c = cen_ref[0, k]
        cen_sc.append((c, c * c * (-GAMMA * LOG2E)))

    racc = [None] * nb
    cacc = [None] * nb

    for bi in range(nb):
        i0 = bi * BLK
        cix = cx[i0:i0 + BLK]
        ciy = cy[i0:i0 + BLK]
        ciz = cz[i0:i0 + BLK]
        bix = bcx[i0:i0 + BLK]
        biy = bcy[i0:i0 + BLK]
        biz = bcz[i0:i0 + BLK]
        sqi = sqc[i0:i0 + BLK]
        mi = mcol[i0:i0 + BLK]
        for bj in range(bi, nb):
            j0 = bj * BLK
            cjx = rx[:, j0:j0 + BLK]
            cjy = ry[:, j0:j0 + BLK]
            cjz = rz[:, j0:j0 + BLK]
            bjx = brx[:, j0:j0 + BLK]
            bjy = bry[:, j0:j0 + BLK]
            bjz = brz[:, j0:j0 + BLK]
            sqj = sqr[:, j0:j0 + BLK]
            mj = mrow[:, j0:j0 + BLK]

            dx = cix - cjx
            dy = ciy - cjy
            dz = ciz - cjz
            d2diff = dx * dx + dy * dy + dz * dz
            dist = jnp.sqrt(d2diff)
            d2m = (sqi + sqj) - 2.0 * (bix * bjx + biy * bjy + biz * bjz)
            valid = (d2m <= CUTOFF * CUTOFF) & (mi > 0.0) & (mj > 0.0)
            if bi == bj:
                ii = lax.broadcasted_iota(jnp.int32, (BLK, BLK), 0)
                jj = lax.broadcasted_iota(jnp.int32, (BLK, BLK), 1)
                valid = valid & (ii != jj)
            w = jnp.where(valid, 1.0, 0.0).astype(f32)

            dm = jnp.where(valid, dist, 1e4)
            A = dm * dm * (-GAMMA * LOG2E)
            Bv = dm * (2.0 * GAMMA * LOG2E)
            rcols = []
            crows = []
            for k in range(RBF_DIM):
                c, s_c = cen_sc[k]
                e = jnp.exp2(Bv * c + A + s_c)
                rcols.append(jnp.sum(e, axis=1, keepdims=True))
                if bi != bj:
                    crows.append(jnp.sum(e, axis=0, keepdims=True))
            rcols.append(jnp.sum(w, axis=1, keepdims=True))
            rblk = jnp.concatenate(rcols, axis=1)
            racc[bi] = rblk if racc[bi] is None else racc[bi] + rblk
            if bi != bj:
                crows.append(jnp.sum(w, axis=0, keepdims=True))
                cblk = jnp.concatenate(crows, axis=0)
                cacc[bj] = cblk if cacc[bj] is None else cacc[bj] + cblk

    srow = jnp.concatenate(racc, axis=0)
    zpad = jnp.zeros((RBF_DIM + 1, BLK), f32)
    ct = jnp.concatenate(
        [zpad if c is None else c for c in cacc], axis=1)
    ctt = jnp.concatenate(
        [ct, jnp.zeros((BLK - RBF_DIM - 1, n), f32)], axis=0)
    stot = srow + ctt.T[:, : RBF_DIM + 1]

    s = stot[:, :RBF_DIM]
    deg = stot[:, RBF_DIM : RBF_DIM + 1]

    vocab_ids = lax.broadcasted_iota(jnp.int32, (n, VOCAB), 1).astype(f32)
    onehot = (tok == vocab_ids).astype(f32)
    x = jnp.dot(onehot, emb_ref[:, :], preferred_element_type=f32)
    layers = ew_ref.shape[0]
    for l in range(layers):
        agg = (jnp.dot(s, ew_ref[l], preferred_element_type=f32)
               + deg * eb_ref[l : l + 1, :])
        h = jnp.dot(agg, w1_ref[l], preferred_element_type=f32)
        h = jnp.maximum(h + b1_ref[l : l + 1, :], 0.0)
        h = (jnp.dot(h, w2_ref[l], preferred_element_type=f32)
             + b2_ref[l : l + 1, :])
        x = x + h
    x = jnp.where(tok != float(PAD), x, 0.0)
    out_ref[0, :, :] = x


def kernel(src_tokens, padded_coordinates, src_distance, src_edge_type,
           embed_table, rbf_centers, edge_W, edge_b, up_W1, up_b1, up_W2,
           up_b2):
    B, N = src_tokens.shape
    E = embed_table.shape[1]
    L = edge_W.shape[0]
    f32 = jnp.float32

    tokf = src_tokens.astype(f32).reshape(B, N, 1)
    coords = padded_coordinates.astype(f32)
    cen = jnp.zeros((8, 128), f32).at[0, :RBF_DIM].set(rbf_centers.astype(f32))
    ew64 = edge_W[:, :RBF_DIM, :].astype(f32)

    xout = pl.pallas_call(
        _fused_kernel,
        grid=(B,),
        in_specs=[
            pl.BlockSpec((1, N, 3), lambda b: (b, 0, 0)),
            pl.BlockSpec((1, N, 1), lambda b: (b, 0, 0)),
            pl.BlockSpec((8, 128), lambda b: (0, 0)),
            pl.BlockSpec((VOCAB, E), lambda b: (0, 0)),
            pl.BlockSpec((L, RBF_DIM, E), lambda b: (0, 0, 0)),
            pl.BlockSpec((L, E), lambda b: (0, 0)),
            pl.BlockSpec((L, E, E), lambda b: (0, 0, 0)),
            pl.BlockSpec((L, E), lambda b: (0, 0)),
            pl.BlockSpec((L, E, E), lambda b: (0, 0, 0)),
            pl.BlockSpec((L, E), lambda b: (0, 0)),
        ],
        out_specs=pl.BlockSpec((1, N, E), lambda b: (b, 0, 0)),
        out_shape=jax.ShapeDtypeStruct((B, N, E), f32),
    )(coords, tokf, cen, embed_table.astype(f32), ew64, edge_b.astype(f32),
      up_W1.astype(f32), up_b1.astype(f32), up_W2.astype(f32),
      up_b2.astype(f32))

    padding_mask = src_tokens == PAD
    return (xout, padding_mask)


# speedup vs baseline: 500.1173x; 1.0208x over previous
import jax
import jax.numpy as jnp
from jax import lax
from jax.experimental import pallas as pl
from jax.experimental.pallas import tpu as pltpu

VOCAB = 128
PAD = 0
RBF_DIM = 64
CUTOFF = 6.0
GAMMA = 10.0
LOG2E = 1.4426950408889634
BLK = 128


def _fused_kernel(coord_ref, tok_ref, tokp_ref, cen_ref, emb_ref, ew_ref,
                  eb_ref, w1_ref, b1_ref, w2_ref, b2_ref, out_ref, s_scr):
    b = pl.program_id(0)
    nsteps = pl.num_programs(0)
    n = coord_ref.shape[1]
    nb = n // BLK
    f32 = jnp.float32

    @pl.when(b > 0)
    def _mlp():
        tok = tokp_ref[0, :, :]
        s = s_scr[:, :RBF_DIM]
        deg = s_scr[:, RBF_DIM : RBF_DIM + 1]
        vocab_ids = lax.broadcasted_iota(jnp.int32, (n, VOCAB), 1).astype(f32)
        onehot = (tok == vocab_ids).astype(f32)
        x = jnp.dot(onehot, emb_ref[:, :], preferred_element_type=f32)
        layers = ew_ref.shape[0]
        for l in range(layers):
            agg = (jnp.dot(s, ew_ref[l], preferred_element_type=f32)
                   + deg * eb_ref[l : l + 1, :])
            h = jnp.dot(agg, w1_ref[l], preferred_element_type=f32)
            h = jnp.maximum(h + b1_ref[l : l + 1, :], 0.0)
            h = (jnp.dot(h, w2_ref[l], preferred_element_type=f32)
                 + b2_ref[l : l + 1, :])
            x = x + h
        x = jnp.where(tok != float(PAD), x, 0.0)
        out_ref[0, :, :] = x

    @pl.when(b < nsteps - 1)
    def _segment_reduce():
        cx = coord_ref[0, :, 0:1]
        cy = coord_ref[0, :, 1:2]
        cz = coord_ref[0, :, 2:3]
        tok = tok_ref[0, :, :]
        mcol = jnp.where(tok != float(PAD), 1.0, 0.0).astype(f32)

        c8 = jnp.concatenate(
            [cx, cy, cz, mcol, jnp.zeros((n, 4), f32)], axis=1)
        r8 = c8.T
        rx = r8[0:1, :]
        ry = r8[1:2, :]
        rz = r8[2:3, :]
        mrow = r8[3:4, :]

        bf = jnp.bfloat16
        bcx = cx.astype(bf).astype(f32)
        bcy = cy.astype(bf).astype(f32)
        bcz = cz.astype(bf).astype(f32)
        brx = rx.astype(bf).astype(f32)
        bry = ry.astype(bf).astype(f32)
        brz = rz.astype(bf).astype(f32)
        sqc = cx * cx + cy * cy + cz * cz
        sqr = rx * rx + ry * ry + rz * rz

        cen_sc = []
        for k in range(RBF_DIM):
            c = cen_ref[0, k]
            cen_sc.append((c, c * c * (-GAMMA * LOG2E)))

        racc = [[None] * (RBF_DIM + 1) for _ in range(nb)]
        cacc = [[None] * (RBF_DIM + 1) for _ in range(nb)]

        for bi in range(nb):
            i0 = bi * BLK
            cix = cx[i0:i0 + BLK]
            ciy = cy[i0:i0 + BLK]
            ciz = cz[i0:i0 + BLK]
            bix = bcx[i0:i0 + BLK]
            biy = bcy[i0:i0 + BLK]
            biz = bcz[i0:i0 + BLK]
            sqi = sqc[i0:i0 + BLK]
            mi = mcol[i0:i0 + BLK]
            for bj in range(bi, nb):
                j0 = bj * BLK
                cjx = rx[:, j0:j0 + BLK]
                cjy = ry[:, j0:j0 + BLK]
                cjz = rz[:, j0:j0 + BLK]
                bjx = brx[:, j0:j0 + BLK]
                bjy = bry[:, j0:j0 + BLK]
                bjz = brz[:, j0:j0 + BLK]
                sqj = sqr[:, j0:j0 + BLK]
                mj = mrow[:, j0:j0 + BLK]

                dx = cix - cjx
                dy = ciy - cjy
                dz = ciz - cjz
                d2diff = dx * dx + dy * dy + dz * dz
                dist = jnp.sqrt(d2diff)
                d2m = (sqi + sqj) - 2.0 * (bix * bjx + biy * bjy + biz * bjz)
                valid = (d2m <= CUTOFF * CUTOFF) & (mi > 0.0) & (mj > 0.0)
                if bi == bj:
                    ii = lax.broadcasted_iota(jnp.int32, (BLK, BLK), 0)
                    jj = lax.broadcasted_iota(jnp.int32, (BLK, BLK), 1)
                    valid = valid & (ii != jj)
                w = jnp.where(valid, 1.0, 0.0).astype(f32)

                dm = jnp.where(valid, dist, 1e4)
                A = dm * dm * (-GAMMA * LOG2E)
                Bv = dm * (2.0 * GAMMA * LOG2E)
                ra = racc[bi]
                ca = cacc[bj]
                for k in range(RBF_DIM):
                    c, s_c = cen_sc[k]
                    e = jnp.exp2(Bv * c + A + s_c)
                    rs = jnp.sum(e, axis=1, keepdims=True)
                    ra[k] = rs if ra[k] is None else ra[k] + rs
                    if bi != bj:
                        cs = jnp.sum(e, axis=0, keepdims=True)
                        ca[k] = cs if ca[k] is None else ca[k] + cs
                rs = jnp.sum(w, axis=1, keepdims=True)
                ra[RBF_DIM] = rs if ra[RBF_DIM] is None else ra[RBF_DIM] + rs
                if bi != bj:
                    cs = jnp.sum(w, axis=0, keepdims=True)
                    ca[RBF_DIM] = (cs if ca[RBF_DIM] is None
                                   else ca[RBF_DIM] + cs)

        srow = jnp.concatenate(
            [jnp.concatenate(ra, axis=1) for ra in racc], axis=0)
        zrow = jnp.zeros((1, BLK), f32)
        ct = jnp.concatenate(
            [jnp.concatenate([zrow if p is None else p for p in ca], axis=0)
             for ca in cacc], axis=1)
        ctt = jnp.concatenate(
            [ct, jnp.zeros((BLK - RBF_DIM - 1, n), f32)], axis=0)
        stot = srow + ctt.T[:, : RBF_DIM + 1]
        s_scr[:, : RBF_DIM + 1] = stot


def kernel(src_tokens, padded_coordinates, src_distance, src_edge_type,
           embed_table, rbf_centers, edge_W, edge_b, up_W1, up_b1, up_W2,
           up_b2):
    B, N = src_tokens.shape
    E = embed_table.shape[1]
    L = edge_W.shape[0]
    f32 = jnp.float32

    tokf = src_tokens.astype(f32).reshape(B, N, 1)
    coords = padded_coordinates.astype(f32)
    cen = jnp.zeros((8, 128), f32).at[0, :RBF_DIM].set(rbf_centers.astype(f32))
    ew64 = edge_W[:, :RBF_DIM, :].astype(f32)

    cur = lambda b: (jnp.minimum(b, B - 1), 0, 0)
    prev = lambda b: (jnp.maximum(b, 1) - 1, 0, 0)
    const2 = lambda b: (0, 0)
    const3 = lambda b: (0, 0, 0)

    xout = pl.pallas_call(
        _fused_kernel,
        grid=(B + 1,),
        in_specs=[
            pl.BlockSpec((1, N, 3), cur),
            pl.BlockSpec((1, N, 1), cur),
            pl.BlockSpec((1, N, 1), prev),
            pl.BlockSpec((8, 128), const2),
            pl.BlockSpec((VOCAB, E), const2),
            pl.BlockSpec((L, RBF_DIM, E), const3),
            pl.BlockSpec((L, E), const2),
            pl.BlockSpec((L, E, E), const3),
            pl.BlockSpec((L, E), const2),
            pl.BlockSpec((L, E, E), const3),
            pl.BlockSpec((L, E), const2),
        ],
        out_specs=pl.BlockSpec((1, N, E), prev),
        out_shape=jax.ShapeDtypeStruct((B, N, E), f32),
        scratch_shapes=[pltpu.VMEM((N, 128), f32)],
    )(coords, tokf, tokf, cen, embed_table.astype(f32), ew64,
      edge_b.astype(f32), up_W1.astype(f32), up_b1.astype(f32),
      up_W2.astype(f32), up_b2.astype(f32))

    padding_mask = src_tokens == PAD
    return (xout, padding_mask)


# speedup vs baseline: 544.6492x; 1.0890x over previous
import jax
import jax.numpy as jnp
from jax import lax
from jax.experimental import pallas as pl
from jax.experimental.pallas import tpu as pltpu

VOCAB = 128
PAD = 0
RBF_DIM = 64
CUTOFF = 6.0
GAMMA = 10.0
LOG2E = 1.4426950408889634
BLK = 128


def _fused_kernel(coord_ref, tok_ref, tokp_ref, cen_ref, emb_ref, ew_ref,
                  eb_ref, w1_ref, b1_ref, w2_ref, b2_ref, out_ref, s_scr):
    b = pl.program_id(0)
    nsteps = pl.num_programs(0)
    n = coord_ref.shape[1]
    nb = n // BLK
    f32 = jnp.float32

    @pl.when(b > 0)
    def _mlp():
        tok = tokp_ref[0, :, :]
        s = s_scr[:, :RBF_DIM]
        deg = s_scr[:, RBF_DIM : RBF_DIM + 1]
        vocab_ids = lax.broadcasted_iota(jnp.int32, (n, VOCAB), 1).astype(f32)
        onehot = (tok == vocab_ids).astype(f32)
        x = jnp.dot(onehot, emb_ref[:, :], preferred_element_type=f32)
        layers = ew_ref.shape[0]
        for l in range(layers):
            agg = (jnp.dot(s, ew_ref[l], preferred_element_type=f32)
                   + deg * eb_ref[l : l + 1, :])
            h = jnp.dot(agg, w1_ref[l], preferred_element_type=f32)
            h = jnp.maximum(h + b1_ref[l : l + 1, :], 0.0)
            h = (jnp.dot(h, w2_ref[l], preferred_element_type=f32)
                 + b2_ref[l : l + 1, :])
            x = x + h
        x = jnp.where(tok != float(PAD), x, 0.0)
        out_ref[0, :, :] = x

    @pl.when(b < nsteps - 1)
    def _segment_reduce():
        cx = coord_ref[0, :, 0:1]
        cy = coord_ref[0, :, 1:2]
        cz = coord_ref[0, :, 2:3]
        tok = tok_ref[0, :, :]
        mcol = jnp.where(tok != float(PAD), 1.0, 0.0).astype(f32)

        c8 = jnp.concatenate(
            [cx, cy, cz, mcol, jnp.zeros((n, 4), f32)], axis=1)
        r8 = c8.T
        rx = r8[0:1, :]
        ry = r8[1:2, :]
        rz = r8[2:3, :]
        mrow = r8[3:4, :]

        bf = jnp.bfloat16
        bcx = cx.astype(bf).astype(f32)
        bcy = cy.astype(bf).astype(f32)
        bcz = cz.astype(bf).astype(f32)
        brx = rx.astype(bf).astype(f32)
        bry = ry.astype(bf).astype(f32)
        brz = rz.astype(bf).astype(f32)
        sqc = cx * cx + cy * cy + cz * cz
        sqr = rx * rx + ry * ry + rz * rz

        cen_sc = []
        for k in range(RBF_DIM):
            c = cen_ref[0, k]
            cen_sc.append((c, c * c * (-GAMMA * LOG2E)))

        ones8 = jnp.ones((BLK, 8), f32)

        racc = [[None] * (RBF_DIM + 1) for _ in range(nb)]
        cacc = [[None] * (RBF_DIM + 1) for _ in range(nb)]

        for bi in range(nb):
            i0 = bi * BLK
            cix = cx[i0:i0 + BLK]
            ciy = cy[i0:i0 + BLK]
            ciz = cz[i0:i0 + BLK]
            bix = bcx[i0:i0 + BLK]
            biy = bcy[i0:i0 + BLK]
            biz = bcz[i0:i0 + BLK]
            sqi = sqc[i0:i0 + BLK]
            mi = mcol[i0:i0 + BLK]
            for bj in range(bi, nb):
                j0 = bj * BLK
                cjx = rx[:, j0:j0 + BLK]
                cjy = ry[:, j0:j0 + BLK]
                cjz = rz[:, j0:j0 + BLK]
                bjx = brx[:, j0:j0 + BLK]
                bjy = bry[:, j0:j0 + BLK]
                bjz = brz[:, j0:j0 + BLK]
                sqj = sqr[:, j0:j0 + BLK]
                mj = mrow[:, j0:j0 + BLK]

                dx = cix - cjx
                dy = ciy - cjy
                dz = ciz - cjz
                d2diff = dx * dx + dy * dy + dz * dz
                dist = jnp.sqrt(d2diff)
                d2m = (sqi + sqj) - 2.0 * (bix * bjx + biy * bjy + biz * bjz)
                valid = (d2m <= CUTOFF * CUTOFF) & (mi > 0.0) & (mj > 0.0)
                if bi == bj:
                    ii = lax.broadcasted_iota(jnp.int32, (BLK, BLK), 0)
                    jj = lax.broadcasted_iota(jnp.int32, (BLK, BLK), 1)
                    valid = valid & (ii != jj)
                w = jnp.where(valid, 1.0, 0.0).astype(f32)

                dm = jnp.where(valid, dist, 1e4)
                A = dm * dm * (-GAMMA * LOG2E)
                Bv = dm * (2.0 * GAMMA * LOG2E)
                ra = racc[bi]
                ca = cacc[bj]
                for k in range(RBF_DIM):
                    c, s_c = cen_sc[k]
                    e = jnp.exp2(Bv * c + A + s_c)
                    rs = jnp.dot(e, ones8,
                                 preferred_element_type=f32)[:, 0:1]
                    ra[k] = rs if ra[k] is None else ra[k] + rs
                    if bi != bj:
                        cs = jnp.sum(e, axis=0, keepdims=True)
                        ca[k] = cs if ca[k] is None else ca[k] + cs
                rs = jnp.sum(w, axis=1, keepdims=True)
                ra[RBF_DIM] = rs if ra[RBF_DIM] is None else ra[RBF_DIM] + rs
                if bi != bj:
                    cs = jnp.sum(w, axis=0, keepdims=True)
                    ca[RBF_DIM] = (cs if ca[RBF_DIM] is None
                                   else ca[RBF_DIM] + cs)

        srow = jnp.concatenate(
            [jnp.concatenate(ra, axis=1) for ra in racc], axis=0)
        zrow = jnp.zeros((1, BLK), f32)
        ct = jnp.concatenate(
            [jnp.concatenate([zrow if p is None else p for p in ca], axis=0)
             for ca in cacc], axis=1)
        ctt = jnp.concatenate(
            [ct, jnp.zeros((BLK - RBF_DIM - 1, n), f32)], axis=0)
        stot = srow + ctt.T[:, : RBF_DIM + 1]
        s_scr[:, : RBF_DIM + 1] = stot


def kernel(src_tokens, padded_coordinates, src_distance, src_edge_type,
           embed_table, rbf_centers, edge_W, edge_b, up_W1, up_b1, up_W2,
           up_b2):
    B, N = src_tokens.shape
    E = embed_table.shape[1]
    L = edge_W.shape[0]
    f32 = jnp.float32

    tokf = src_tokens.astype(f32).reshape(B, N, 1)
    coords = padded_coordinates.astype(f32)
    cen = jnp.zeros((8, 128), f32).at[0, :RBF_DIM].set(rbf_centers.astype(f32))
    ew64 = edge_W[:, :RBF_DIM, :].astype(f32)

    cur = lambda b: (jnp.minimum(b, B - 1), 0, 0)
    prev = lambda b: (jnp.maximum(b, 1) - 1, 0, 0)
    const2 = lambda b: (0, 0)
    const3 = lambda b: (0, 0, 0)

    xout = pl.pallas_call(
        _fused_kernel,
        grid=(B + 1,),
        in_specs=[
            pl.BlockSpec((1, N, 3), cur),
            pl.BlockSpec((1, N, 1), cur),
            pl.BlockSpec((1, N, 1), prev),
            pl.BlockSpec((8, 128), const2),
            pl.BlockSpec((VOCAB, E), const2),
            pl.BlockSpec((L, RBF_DIM, E), const3),
            pl.BlockSpec((L, E), const2),
            pl.BlockSpec((L, E, E), const3),
            pl.BlockSpec((L, E), const2),
            pl.BlockSpec((L, E, E), const3),
            pl.BlockSpec((L, E), const2),
        ],
        out_specs=pl.BlockSpec((1, N, E), prev),
        out_shape=jax.ShapeDtypeStruct((B, N, E), f32),
        scratch_shapes=[pltpu.VMEM((N, 128), f32)],
    )(coords, tokf, tokf, cen, embed_table.astype(f32), ew64,
      edge_b.astype(f32), up_W1.astype(f32), up_b1.astype(f32),
      up_W2.astype(f32), up_b2.astype(f32))

    padding_mask = src_tokens == PAD
    return (xout, padding_mask)


# speedup vs baseline: 550.6348x; 1.0110x over previous
import jax
import jax.numpy as jnp
from jax import lax
from jax.experimental import pallas as pl
from jax.experimental.pallas import tpu as pltpu

VOCAB = 128
PAD = 0
RBF_DIM = 64
CUTOFF = 6.0
GAMMA = 10.0
LOG2E = 1.4426950408889634
BLK = 128


def _fused_kernel(coord_ref, tok_ref, tokp_ref, cen_ref, emb_ref, ew_ref,
                  eb_ref, w1_ref, b1_ref, w2_ref, b2_ref, out_ref, s_scr):
    b = pl.program_id(0)
    nsteps = pl.num_programs(0)
    n = coord_ref.shape[1]
    nb = n // BLK
    f32 = jnp.float32

    @pl.when(b > 0)
    def _mlp():
        tok = tokp_ref[0, :, :]
        s = s_scr[:, :RBF_DIM]
        deg = s_scr[:, RBF_DIM : RBF_DIM + 1]
        vocab_ids = lax.broadcasted_iota(jnp.int32, (n, VOCAB), 1).astype(f32)
        onehot = (tok == vocab_ids).astype(f32)
        x = jnp.dot(onehot, emb_ref[:, :], preferred_element_type=f32)
        layers = ew_ref.shape[0]
        for l in range(layers):
            agg = (jnp.dot(s, ew_ref[l, :RBF_DIM, :],
                           preferred_element_type=f32)
                   + deg * eb_ref[l : l + 1, :])
            h = jnp.dot(agg, w1_ref[l], preferred_element_type=f32)
            h = jnp.maximum(h + b1_ref[l : l + 1, :], 0.0)
            h = (jnp.dot(h, w2_ref[l], preferred_element_type=f32)
                 + b2_ref[l : l + 1, :])
            x = x + h
        x = jnp.where(tok != float(PAD), x, 0.0)
        out_ref[0, :, :] = x

    @pl.when(b < nsteps - 1)
    def _segment_reduce():
        cx = coord_ref[0, :, 0:1]
        cy = coord_ref[0, :, 1:2]
        cz = coord_ref[0, :, 2:3]
        tok = tok_ref[0, :, :]
        mcol = jnp.where(tok != float(PAD), 1.0, 0.0).astype(f32)

        c8 = jnp.concatenate(
            [cx, cy, cz, mcol, jnp.zeros((n, 4), f32)], axis=1)
        r8 = c8.T
        rx = r8[0:1, :]
        ry = r8[1:2, :]
        rz = r8[2:3, :]
        mrow = r8[3:4, :]

        bf = jnp.bfloat16
        bcx = cx.astype(bf).astype(f32)
        bcy = cy.astype(bf).astype(f32)
        bcz = cz.astype(bf).astype(f32)
        brx = rx.astype(bf).astype(f32)
        bry = ry.astype(bf).astype(f32)
        brz = rz.astype(bf).astype(f32)
        sqc = cx * cx + cy * cy + cz * cz
        sqr = rx * rx + ry * ry + rz * rz

        cen_sc = []
        for k in range(RBF_DIM):
            c = cen_ref[0, k]
            cen_sc.append((c, c * c * (-GAMMA * LOG2E)))

        ones8 = jnp.ones((BLK, 8), f32)

        racc = [[None] * (RBF_DIM + 1) for _ in range(nb)]
        cacc = [[None] * (RBF_DIM + 1) for _ in range(nb)]

        for bi in range(nb):
            i0 = bi * BLK
            cix = cx[i0:i0 + BLK]
            ciy = cy[i0:i0 + BLK]
            ciz = cz[i0:i0 + BLK]
            bix = bcx[i0:i0 + BLK]
            biy = bcy[i0:i0 + BLK]
            biz = bcz[i0:i0 + BLK]
            sqi = sqc[i0:i0 + BLK]
            mi = mcol[i0:i0 + BLK]
            for bj in range(bi, nb):
                j0 = bj * BLK
                cjx = rx[:, j0:j0 + BLK]
                cjy = ry[:, j0:j0 + BLK]
                cjz = rz[:, j0:j0 + BLK]
                bjx = brx[:, j0:j0 + BLK]
                bjy = bry[:, j0:j0 + BLK]
                bjz = brz[:, j0:j0 + BLK]
                sqj = sqr[:, j0:j0 + BLK]
                mj = mrow[:, j0:j0 + BLK]

                dx = cix - cjx
                dy = ciy - cjy
                dz = ciz - cjz
                d2diff = dx * dx + dy * dy + dz * dz
                dist = jnp.sqrt(d2diff)
                d2m = (sqi + sqj) - 2.0 * (bix * bjx + biy * bjy + biz * bjz)
                valid = (d2m <= CUTOFF * CUTOFF) & (mi > 0.0) & (mj > 0.0)
                if bi == bj:
                    ii = lax.broadcasted_iota(jnp.int32, (BLK, BLK), 0)
                    jj = lax.broadcasted_iota(jnp.int32, (BLK, BLK), 1)
                    valid = valid & (ii != jj)
                w = jnp.where(valid, 1.0, 0.0).astype(f32)

                dm = jnp.where(valid, dist, 1e4)
                A = dm * dm * (-GAMMA * LOG2E)
                Bv = dm * (2.0 * GAMMA * LOG2E)
                ra = racc[bi]
                ca = cacc[bj]
                for k in range(RBF_DIM):
                    c, s_c = cen_sc[k]
                    e = jnp.exp2(Bv * c + A + s_c)
                    rs = jnp.dot(e, ones8,
                                 preferred_element_type=f32)[:, 0:1]
                    ra[k] = rs if ra[k] is None else ra[k] + rs
                    if bi != bj:
                        cs = jnp.sum(e, axis=0, keepdims=True)
                        ca[k] = cs if ca[k] is None else ca[k] + cs
                rs = jnp.sum(w, axis=1, keepdims=True)
                ra[RBF_DIM] = rs if ra[RBF_DIM] is None else ra[RBF_DIM] + rs
                if bi != bj:
                    cs = jnp.sum(w, axis=0, keepdims=True)
                    ca[RBF_DIM] = (cs if ca[RBF_DIM] is None
                                   else ca[RBF_DIM] + cs)

        srow = jnp.concatenate(
            [jnp.concatenate(ra, axis=1) for ra in racc], axis=0)
        zrow = jnp.zeros((1, BLK), f32)
        ct = jnp.concatenate(
            [jnp.concatenate([zrow if p is None else p for p in ca], axis=0)
             for ca in cacc], axis=1)
        ctt = jnp.concatenate(
            [ct, jnp.zeros((BLK - RBF_DIM - 1, n), f32)], axis=0)
        stot = srow + ctt.T[:, : RBF_DIM + 1]
        s_scr[:, : RBF_DIM + 1] = stot


def kernel(src_tokens, padded_coordinates, src_distance, src_edge_type,
           embed_table, rbf_centers, edge_W, edge_b, up_W1, up_b1, up_W2,
           up_b2):
    B, N = src_tokens.shape
    E = embed_table.shape[1]
    L = edge_W.shape[0]
    f32 = jnp.float32

    tokf = src_tokens.astype(f32).reshape(B, N, 1)
    coords = padded_coordinates.astype(f32)
    cen = rbf_centers.astype(f32).reshape(1, RBF_DIM)

    cur = lambda b: (jnp.minimum(b, B - 1), 0, 0)
    prev = lambda b: (jnp.maximum(b, 1) - 1, 0, 0)
    const2 = lambda b: (0, 0)
    const3 = lambda b: (0, 0, 0)

    xout = pl.pallas_call(
        _fused_kernel,
        grid=(B + 1,),
        in_specs=[
            pl.BlockSpec((1, N, 3), cur),
            pl.BlockSpec((1, N, 1), cur),
            pl.BlockSpec((1, N, 1), prev),
            pl.BlockSpec((1, RBF_DIM), const2),
            pl.BlockSpec((VOCAB, E), const2),
            pl.BlockSpec((L, edge_W.shape[1], E), const3),
            pl.BlockSpec((L, E), const2),
            pl.BlockSpec((L, E, E), const3),
            pl.BlockSpec((L, E), const2),
            pl.BlockSpec((L, E, E), const3),
            pl.BlockSpec((L, E), const2),
        ],
        out_specs=pl.BlockSpec((1, N, E), prev),
        out_shape=jax.ShapeDtypeStruct((B, N, E), f32),
        scratch_shapes=[pltpu.VMEM((N, 128), f32)],
    )(coords, tokf, tokf, cen, embed_table.astype(f32), edge_W.astype(f32),
      edge_b.astype(f32), up_W1.astype(f32), up_b1.astype(f32),
      up_W2.astype(f32), up_b2.astype(f32))

    padding_mask = src_tokens == PAD
    return (xout, padding_mask)
